# baseline (device time: 76098 ns/iter reference)
import functools

import jax
import jax.numpy as jnp
from jax import lax
from jax.experimental import pallas as pl
from jax.experimental.pallas import tpu as pltpu

N_DEV = 16


def _gelu(y):
    c = 0.7978845608028654
    return 0.5 * y * (1.0 + jnp.tanh(c * (y + 0.044715 * y * y * y)))


def kernel(x, w_mat):
    m, k = x.shape
    _, n = w_mat.shape
    m_out = m // N_DEV

    def body(x_ref, w_ref, out_ref, acc_ref, comm_ref, send_sems, recv_sems):
        my = lax.axis_index("i")
        left = jnp.mod(my - 1, N_DEV)
        right = jnp.mod(my + 1, N_DEV)

        barrier = pltpu.get_barrier_semaphore()
        for nbr in (left, right):
            pl.semaphore_signal(
                barrier, inc=1, device_id=(nbr,),
                device_id_type=pl.DeviceIdType.MESH,
            )
        pl.semaphore_wait(barrier, 2)

        acc_ref[:, :] = jnp.dot(
            x_ref[:, :], w_ref[:, :], preferred_element_type=jnp.float32
        )

        for s in range(N_DEV - 1):
            send_chunk = jnp.mod(my - s - 1, N_DEV)
            rdma = pltpu.make_async_remote_copy(
                src_ref=acc_ref.at[pl.ds(send_chunk * m_out, m_out), :],
                dst_ref=comm_ref.at[s],
                send_sem=send_sems.at[s],
                recv_sem=recv_sems.at[s],
                device_id=(right,),
                device_id_type=pl.DeviceIdType.MESH,
            )
            rdma.start()
            rdma.wait()
            recv_chunk = jnp.mod(my - s - 2, N_DEV)
            row = recv_chunk * m_out
            acc_ref[pl.ds(row, m_out), :] = (
                acc_ref[pl.ds(row, m_out), :] + comm_ref[s, :, :]
            )

        out_ref[:, :] = _gelu(acc_ref[pl.ds(my * m_out, m_out), :])

        @functools.partial(
            pl.run_scoped, second_barrier=pltpu.SemaphoreType.REGULAR
        )
        def _(second_barrier):
            for nbr in (left, right):
                pl.semaphore_signal(
                    second_barrier, inc=1, device_id=(nbr,),
                    device_id_type=pl.DeviceIdType.MESH,
                )
            pl.semaphore_wait(second_barrier, 2)

    return pl.pallas_call(
        body,
        out_shape=jax.ShapeDtypeStruct((m_out, n), jnp.float32),
        in_specs=[
            pl.BlockSpec(memory_space=pltpu.VMEM),
            pl.BlockSpec(memory_space=pltpu.VMEM),
        ],
        out_specs=pl.BlockSpec(memory_space=pltpu.VMEM),
        scratch_shapes=[
            pltpu.VMEM((m, n), jnp.float32),
            pltpu.VMEM((N_DEV - 1, m_out, n), jnp.float32),
            pltpu.SemaphoreType.DMA((N_DEV - 1,)),
            pltpu.SemaphoreType.DMA((N_DEV - 1,)),
        ],
        compiler_params=pltpu.CompilerParams(collective_id=0),
    )(x, w_mat)


# device time: 75214 ns/iter; 1.0118x vs baseline; 1.0118x over previous
import functools

import jax
import jax.numpy as jnp
from jax import lax
from jax.experimental import pallas as pl
from jax.experimental.pallas import tpu as pltpu

N_DEV = 16


def _gelu(y):
    c = 0.7978845608028654
    return 0.5 * y * (1.0 + jnp.tanh(c * (y + 0.044715 * y * y * y)))


def kernel(x, w_mat):
    m, k = x.shape
    _, n = w_mat.shape
    m_out = m // N_DEV
    nh = n // 2

    def body(x_ref, w_ref, out_ref, acc_ref,
             fcomm_ref, bcomm_ref, fsend_sems, frecv_sems, bsend_sems, brecv_sems):
        my = lax.axis_index("i")
        left = jnp.mod(my - 1, N_DEV)
        right = jnp.mod(my + 1, N_DEV)

        barrier = pltpu.get_barrier_semaphore()
        for nbr in (left, right):
            pl.semaphore_signal(
                barrier, inc=1, device_id=(nbr,),
                device_id_type=pl.DeviceIdType.MESH,
            )
        pl.semaphore_wait(barrier, 2)

        acc_ref[:, :] = jnp.dot(
            x_ref[:, :], w_ref[:, :], preferred_element_type=jnp.float32
        )

        prev = None
        for s in range(N_DEV - 1):
            f_send = jnp.mod(my - s - 1, N_DEV)
            b_send = jnp.mod(my + s + 1, N_DEV)
            fr = pltpu.make_async_remote_copy(
                src_ref=acc_ref.at[pl.ds(f_send * m_out, m_out), pl.ds(0, nh)],
                dst_ref=fcomm_ref.at[s],
                send_sem=fsend_sems.at[s],
                recv_sem=frecv_sems.at[s],
                device_id=(right,),
                device_id_type=pl.DeviceIdType.MESH,
            )
            br = pltpu.make_async_remote_copy(
                src_ref=acc_ref.at[pl.ds(b_send * m_out, m_out), pl.ds(nh, nh)],
                dst_ref=bcomm_ref.at[s],
                send_sem=bsend_sems.at[s],
                recv_sem=brecv_sems.at[s],
                device_id=(left,),
                device_id_type=pl.DeviceIdType.MESH,
            )
            fr.start()
            br.start()
            if prev is not None:
                prev[0].wait_send()
                prev[1].wait_send()
            prev = (fr, br)

            fr.wait_recv()
            f_recv = jnp.mod(my - s - 2, N_DEV)
            frow = f_recv * m_out
            acc_ref[pl.ds(frow, m_out), pl.ds(0, nh)] = (
                acc_ref[pl.ds(frow, m_out), pl.ds(0, nh)] + fcomm_ref[s, :, :]
            )
            br.wait_recv()
            b_recv = jnp.mod(my + s + 2, N_DEV)
            brow = b_recv * m_out
            acc_ref[pl.ds(brow, m_out), pl.ds(nh, nh)] = (
                acc_ref[pl.ds(brow, m_out), pl.ds(nh, nh)] + bcomm_ref[s, :, :]
            )
        prev[0].wait_send()
        prev[1].wait_send()

        out_ref[:, :] = _gelu(acc_ref[pl.ds(my * m_out, m_out), :])

        @functools.partial(
            pl.run_scoped, second_barrier=pltpu.SemaphoreType.REGULAR
        )
        def _(second_barrier):
            for nbr in (left, right):
                pl.semaphore_signal(
                    second_barrier, inc=1, device_id=(nbr,),
                    device_id_type=pl.DeviceIdType.MESH,
                )
            pl.semaphore_wait(second_barrier, 2)

    return pl.pallas_call(
        body,
        out_shape=jax.ShapeDtypeStruct((m_out, n), jnp.float32),
        in_specs=[
            pl.BlockSpec(memory_space=pltpu.VMEM),
            pl.BlockSpec(memory_space=pltpu.VMEM),
        ],
        out_specs=pl.BlockSpec(memory_space=pltpu.VMEM),
        scratch_shapes=[
            pltpu.VMEM((m, n), jnp.float32),
            pltpu.VMEM((N_DEV - 1, m_out, nh), jnp.float32),
            pltpu.VMEM((N_DEV - 1, m_out, nh), jnp.float32),
            pltpu.SemaphoreType.DMA((N_DEV - 1,)),
            pltpu.SemaphoreType.DMA((N_DEV - 1,)),
            pltpu.SemaphoreType.DMA((N_DEV - 1,)),
            pltpu.SemaphoreType.DMA((N_DEV - 1,)),
        ],
        compiler_params=pltpu.CompilerParams(collective_id=0),
    )(x, w_mat)


# device time: 60184 ns/iter; 1.2644x vs baseline; 1.2497x over previous
import functools

import jax
import jax.numpy as jnp
import numpy as np
from jax import lax
from jax.experimental import pallas as pl
from jax.experimental.pallas import tpu as pltpu

N_DEV = 16

_RING = np.array([0, 1, 5, 9, 13, 14, 10, 6, 2, 3, 7, 11, 15, 12, 8, 4])
_INV = np.zeros(N_DEV, dtype=np.int32)
_INV[_RING] = np.arange(N_DEV, dtype=np.int32)


def _gelu(y):
    c = 0.7978845608028654
    return 0.5 * y * (1.0 + jnp.tanh(c * (y + 0.044715 * y * y * y)))


def kernel(x, w_mat):
    m, k = x.shape
    _, n = w_mat.shape
    m_out = m // N_DEV
    nh = n // 2

    ring = jnp.asarray(_RING, dtype=jnp.int32)
    my = lax.axis_index("i")
    r = jnp.asarray(_INV, dtype=jnp.int32)[my]
    s_arange = jnp.arange(N_DEV - 1, dtype=jnp.int32)
    meta = jnp.concatenate([
        jnp.stack([ring[jnp.mod(r + 1, N_DEV)], ring[jnp.mod(r - 1, N_DEV)]]),
        ring[jnp.mod(r - 1 - s_arange, N_DEV)] * m_out,
        ring[jnp.mod(r - 2 - s_arange, N_DEV)] * m_out,
        ring[jnp.mod(r + 1 + s_arange, N_DEV)] * m_out,
        ring[jnp.mod(r + 2 + s_arange, N_DEV)] * m_out,
    ]).astype(jnp.int32)

    def body(meta_ref, x_ref, w_ref, out_ref, acc_ref,
             fcomm_ref, bcomm_ref, fsend_sems, frecv_sems, bsend_sems, brecv_sems):
        right = meta_ref[0]
        left = meta_ref[1]

        barrier = pltpu.get_barrier_semaphore()
        for nbr in (left, right):
            pl.semaphore_signal(
                barrier, inc=1, device_id=(nbr,),
                device_id_type=pl.DeviceIdType.MESH,
            )
        pl.semaphore_wait(barrier, 2)

        acc_ref[:, :] = jnp.dot(
            x_ref[:, :], w_ref[:, :], preferred_element_type=jnp.float32
        )

        prev = None
        for s in range(N_DEV - 1):
            fr = pltpu.make_async_remote_copy(
                src_ref=acc_ref.at[pl.ds(pl.multiple_of(meta_ref[2 + s], m_out), m_out), pl.ds(0, nh)],
                dst_ref=fcomm_ref.at[s],
                send_sem=fsend_sems.at[s],
                recv_sem=frecv_sems.at[s],
                device_id=(right,),
                device_id_type=pl.DeviceIdType.MESH,
            )
            br = pltpu.make_async_remote_copy(
                src_ref=acc_ref.at[pl.ds(pl.multiple_of(meta_ref[32 + s], m_out), m_out), pl.ds(nh, nh)],
                dst_ref=bcomm_ref.at[s],
                send_sem=bsend_sems.at[s],
                recv_sem=brecv_sems.at[s],
                device_id=(left,),
                device_id_type=pl.DeviceIdType.MESH,
            )
            fr.start()
            br.start()
            if prev is not None:
                prev[0].wait_send()
                prev[1].wait_send()
            prev = (fr, br)

            fr.wait_recv()
            frow = pl.multiple_of(meta_ref[17 + s], m_out)
            acc_ref[pl.ds(frow, m_out), pl.ds(0, nh)] = (
                acc_ref[pl.ds(frow, m_out), pl.ds(0, nh)] + fcomm_ref[s, :, :]
            )
            br.wait_recv()
            brow = pl.multiple_of(meta_ref[47 + s], m_out)
            acc_ref[pl.ds(brow, m_out), pl.ds(nh, nh)] = (
                acc_ref[pl.ds(brow, m_out), pl.ds(nh, nh)] + bcomm_ref[s, :, :]
            )
        prev[0].wait_send()
        prev[1].wait_send()

        myrow = pl.multiple_of(meta_ref[17 + N_DEV - 2], m_out)
        out_ref[:, :] = _gelu(acc_ref[pl.ds(myrow, m_out), :])

        @functools.partial(
            pl.run_scoped, second_barrier=pltpu.SemaphoreType.REGULAR
        )
        def _(second_barrier):
            for nbr in (left, right):
                pl.semaphore_signal(
                    second_barrier, inc=1, device_id=(nbr,),
                    device_id_type=pl.DeviceIdType.MESH,
                )
            pl.semaphore_wait(second_barrier, 2)

    return pl.pallas_call(
        body,
        out_shape=jax.ShapeDtypeStruct((m_out, n), jnp.float32),
        in_specs=[
            pl.BlockSpec(memory_space=pltpu.SMEM),
            pl.BlockSpec(memory_space=pltpu.VMEM),
            pl.BlockSpec(memory_space=pltpu.VMEM),
        ],
        out_specs=pl.BlockSpec(memory_space=pltpu.VMEM),
        scratch_shapes=[
            pltpu.VMEM((m, n), jnp.float32),
            pltpu.VMEM((N_DEV - 1, m_out, nh), jnp.float32),
            pltpu.VMEM((N_DEV - 1, m_out, nh), jnp.float32),
            pltpu.SemaphoreType.DMA((N_DEV - 1,)),
            pltpu.SemaphoreType.DMA((N_DEV - 1,)),
            pltpu.SemaphoreType.DMA((N_DEV - 1,)),
            pltpu.SemaphoreType.DMA((N_DEV - 1,)),
        ],
        compiler_params=pltpu.CompilerParams(collective_id=0),
    )(meta, x, w_mat)
